# Initial kernel scaffold; baseline (speedup 1.0000x reference)
import jax
import jax.numpy as jnp
from jax.experimental import pallas as pl


def kernel(asset_class, borrower_type, rate_type, amort_type, continuous_features, ac_table, bt_table, rt_table, at_table, W1, b1, W2, b2, Wo, bo):
    raise NotImplementedError("write your pallas kernel here")


# speedup vs baseline: 12.0159x; 12.0159x over previous
import functools

import jax
import jax.numpy as jnp
from jax import lax
from jax.experimental import pallas as pl
from jax.experimental.pallas import tpu as pltpu

B = 16384
D = 128
BB = 2048
G = B // BB


def _dot(a, b):
    return lax.dot_general(a, b, (((1,), (0,)), ((), ())),
                           preferred_element_type=jnp.float32)


def _dot_t(a, b):
    return lax.dot_general(a, b, (((0,), (0,)), ((), ())),
                           preferred_element_type=jnp.float32)


def _body(ac_ref, bt_ref, rt_ref, at_ref, x_ref,
          ac_t_ref, bt_t_ref, rt_t_ref, at_t_ref,
          w1_ref, b1_ref, w2_ref, b2_ref, wo_ref, bo_ref, out_ref):
    f32 = jnp.float32
    wo = wo_ref[...]

    p_ac = _dot(ac_t_ref[...], wo[0:32, :])
    p_bt = _dot(bt_t_ref[...], wo[32:64, :])
    p_rt = _dot(rt_t_ref[...], wo[64:80, :])
    p_at = _dot(at_t_ref[...], wo[80:96, :])

    def onehot(idx_row, n):
        col = lax.broadcasted_iota(jnp.int32, (n, BB), 0)
        return (col == idx_row).astype(f32)

    emb = _dot_t(onehot(ac_ref[0], 4), p_ac)
    emb += _dot_t(onehot(bt_ref[0], 4), p_bt)
    emb += _dot_t(onehot(rt_ref[0], 2), p_rt)
    emb += _dot_t(onehot(at_ref[0], 3), p_at)

    w2p = _dot(w2_ref[...], wo[96:128, :])
    c0 = _dot(b2_ref[...], wo[96:128, :]) + bo_ref[...]
    h = jnp.maximum(_dot(x_ref[...], w1_ref[...]) + b1_ref[...], 0.0)
    out_ref[...] = emb + _dot(h, w2p) + c0


@jax.jit
def kernel(asset_class, borrower_type, rate_type, amort_type,
           continuous_features, ac_table, bt_table, rt_table, at_table,
           W1, b1, W2, b2, Wo, bo):
    n_cont = continuous_features.shape[1]
    idx3 = lambda a: a.reshape(G, 1, BB)
    idx_spec = pl.BlockSpec((1, 1, BB), lambda i: (i, 0, 0))
    full = lambda shape: pl.BlockSpec(shape, lambda i: tuple(0 for _ in shape))

    out = pl.pallas_call(
        _body,
        grid=(G,),
        in_specs=[idx_spec, idx_spec, idx_spec, idx_spec,
                  pl.BlockSpec((BB, n_cont), lambda i: (i, 0)),
                  full((4, 32)), full((4, 32)), full((2, 16)), full((3, 16)),
                  full((n_cont, 64)), full((1, 64)),
                  full((64, 32)), full((1, 32)),
                  full((128, 128)), full((1, 128))],
        out_specs=pl.BlockSpec((BB, D), lambda i: (i, 0)),
        out_shape=jax.ShapeDtypeStruct((B, D), jnp.float32),
        compiler_params=pltpu.CompilerParams(
            dimension_semantics=("arbitrary",)),
    )(idx3(asset_class), idx3(borrower_type), idx3(rate_type),
      idx3(amort_type), continuous_features,
      ac_table, bt_table, rt_table, at_table,
      W1, b1.reshape(1, 64), W2, b2.reshape(1, 32), Wo, bo.reshape(1, 128))
    return out
